# baseline (speedup 1.0000x reference)
import functools

import jax
import jax.numpy as jnp
from jax import lax
from jax.experimental import pallas as pl
from jax.experimental.pallas import tpu as pltpu
from jax.experimental.pallas import tpu_sc as plsc

HIDDEN = 768
FF = 3072
E = 8
TOP_K = 2
T = 2048

BLK = 256
G = (T * TOP_K) // BLK + E
CAP = G * BLK

NW = 32
TPW = T // NW


def _router_body(x_ref, wg_ref, bg_ref, idx_ref, gates_ref):
    x = x_ref[...]
    logits = jax.lax.dot_general(
        x, wg_ref[...], (((1,), (0,)), ((), ())),
        preferred_element_type=jnp.float32) + bg_ref[...][None, :]
    col = jax.lax.broadcasted_iota(jnp.int32, (T, E), 1)
    m1 = jnp.max(logits, axis=1, keepdims=True)
    i1 = jnp.min(jnp.where(logits == m1, col, E), axis=1, keepdims=True)
    masked = jnp.where(col == i1, -jnp.inf, logits)
    m2 = jnp.max(masked, axis=1, keepdims=True)
    i2 = jnp.min(jnp.where(masked == m2, col, E), axis=1, keepdims=True)
    e2 = jnp.exp(m2 - m1)
    g1 = 1.0 / (1.0 + e2)
    g2 = e2 * g1
    idx_ref[...] = jnp.concatenate([i1, i2], axis=1)
    gates_ref[...] = jnp.concatenate([g1, g2], axis=1)


def _ffn_body(be_ref, na_ref, xs_ref, gr_ref, w1_ref, b1_ref, w2_ref,
              b2_ref, ys_ref):
    g = pl.program_id(0)

    @pl.when(g < na_ref[0])
    def _():
        h = jax.nn.gelu(jax.lax.dot_general(
            xs_ref[...], w1_ref[0], (((1,), (0,)), ((), ())),
            preferred_element_type=jnp.float32) + b1_ref[0])
        eo = jax.lax.dot_general(
            h, w2_ref[0], (((1,), (0,)), ((), ())),
            preferred_element_type=jnp.float32) + b2_ref[0]
        ys_ref[...] = eo * gr_ref[...]


def _make_dispatch():
    mesh = plsc.VectorSubcoreMesh(core_axis_name="c", subcore_axis_name="s")

    @functools.partial(
        pl.kernel, mesh=mesh,
        out_type=[
            jax.ShapeDtypeStruct((CAP, HIDDEN), jnp.float32),
            jax.ShapeDtypeStruct((CAP,), jnp.float32),
        ],
        scratch_types=[
            pltpu.VMEM((TPW, HIDDEN), jnp.float32),
            pltpu.VMEM((TPW,), jnp.int32),
            pltpu.VMEM((TPW,), jnp.int32),
            pltpu.VMEM((TPW,), jnp.float32),
            pltpu.VMEM((TPW,), jnp.float32),
            pltpu.SemaphoreType.DMA,
        ],
    )
    def dispatch(x_hbm, d0_hbm, d1_hbm, g0_hbm, g1_hbm, xs_hbm, gr_hbm,
                 xch, i0, i1, g0, g1, sem):
        wid = lax.axis_index("s") * 2 + lax.axis_index("c")
        base = wid * TPW
        pltpu.sync_copy(x_hbm.at[pl.ds(base, TPW)], xch)
        pltpu.sync_copy(d0_hbm.at[wid], i0)
        pltpu.sync_copy(d1_hbm.at[wid], i1)
        pltpu.sync_copy(g0_hbm.at[wid], g0)
        pltpu.sync_copy(g1_hbm.at[wid], g1)
        c0 = pltpu.async_copy(xch, xs_hbm.at[i0], sem)
        c1 = pltpu.async_copy(xch, xs_hbm.at[i1], sem)
        c2 = pltpu.async_copy(g0, gr_hbm.at[i0], sem)
        c3 = pltpu.async_copy(g1, gr_hbm.at[i1], sem)
        c0.wait()
        c1.wait()
        c2.wait()
        c3.wait()

    return dispatch


def _make_combine():
    mesh = plsc.VectorSubcoreMesh(core_axis_name="c", subcore_axis_name="s")

    @functools.partial(
        pl.kernel, mesh=mesh,
        out_type=jax.ShapeDtypeStruct((T, HIDDEN), jnp.float32),
        scratch_types=[
            pltpu.VMEM((TPW, HIDDEN), jnp.float32),
            pltpu.VMEM((TPW, HIDDEN), jnp.float32),
            pltpu.VMEM((TPW,), jnp.int32),
            pltpu.VMEM((TPW,), jnp.int32),
            pltpu.SemaphoreType.DMA,
        ],
    )
    def combine(ys_hbm, d0_hbm, d1_hbm, out_hbm, a, b, i0, i1, sem):
        wid = lax.axis_index("s") * 2 + lax.axis_index("c")
        base = wid * TPW
        pltpu.sync_copy(d0_hbm.at[wid], i0)
        pltpu.sync_copy(d1_hbm.at[wid], i1)
        c0 = pltpu.async_copy(ys_hbm.at[i0], a, sem)
        c1 = pltpu.async_copy(ys_hbm.at[i1], b, sem)
        c0.wait()
        c1.wait()

        def row_body(r, carry):
            for j in range(HIDDEN // 16):
                sl = pl.ds(j * 16, 16)
                a[r, sl] = a[r, sl] + b[r, sl]
            return carry

        lax.fori_loop(0, TPW, row_body, 0)
        pltpu.sync_copy(a, out_hbm.at[pl.ds(base, TPW)])

    return combine


def kernel(x, Wg, bg, W1, b1, W2, b2):
    B, S, H = x.shape
    x_flat = x.reshape(-1, H)

    top_idx, gates = pl.pallas_call(
        _router_body,
        grid=(1,),
        in_specs=[
            pl.BlockSpec((T, HIDDEN), lambda i: (0, 0)),
            pl.BlockSpec((HIDDEN, E), lambda i: (0, 0)),
            pl.BlockSpec((E,), lambda i: (0,)),
        ],
        out_specs=[
            pl.BlockSpec((T, TOP_K), lambda i: (0, 0)),
            pl.BlockSpec((T, TOP_K), lambda i: (0, 0)),
        ],
        out_shape=[
            jax.ShapeDtypeStruct((T, TOP_K), jnp.int32),
            jax.ShapeDtypeStruct((T, TOP_K), jnp.float32),
        ],
    )(x_flat, Wg, bg)

    oh = top_idx[..., None] == jnp.arange(E)[None, None, :]
    c = oh.sum(1).astype(jnp.int32)
    incl = jnp.cumsum(c, axis=0)
    excl = incl - c
    counts = incl[-1]
    blocks_e = (counts + BLK - 1) // BLK
    cumB = jnp.cumsum(blocks_e)
    row_start = (cumB - blocks_e) * BLK
    base = row_start[None, :] + excl
    dest0 = jnp.sum(jnp.where(oh[:, 0], base, 0), axis=-1).astype(jnp.int32)
    dest1 = jnp.sum(jnp.where(oh[:, 1], base, 0), axis=-1).astype(jnp.int32)
    n_active = cumB[-1]
    g_idx = jnp.arange(G, dtype=jnp.int32)
    be = jnp.clip((g_idx[:, None] >= cumB[None, :]).sum(-1), 0, E - 1)
    be = jnp.where(g_idx < n_active, be, be[n_active - 1]).astype(jnp.int32)
    xs, gate_rows = _make_dispatch()(
        x_flat, dest0.reshape(NW, TPW), dest1.reshape(NW, TPW),
        gates[:, 0].reshape(NW, TPW), gates[:, 1].reshape(NW, TPW))

    ys = pl.pallas_call(
        _ffn_body,
        grid_spec=pltpu.PrefetchScalarGridSpec(
            num_scalar_prefetch=2,
            grid=(G,),
            in_specs=[
                pl.BlockSpec((BLK, HIDDEN), lambda g, be_r, na_r: (g, 0)),
                pl.BlockSpec((BLK, 1), lambda g, be_r, na_r: (g, 0)),
                pl.BlockSpec((1, HIDDEN, FF),
                             lambda g, be_r, na_r: (be_r[g], 0, 0)),
                pl.BlockSpec((1, 1, FF),
                             lambda g, be_r, na_r: (be_r[g], 0, 0)),
                pl.BlockSpec((1, FF, HIDDEN),
                             lambda g, be_r, na_r: (be_r[g], 0, 0)),
                pl.BlockSpec((1, 1, HIDDEN),
                             lambda g, be_r, na_r: (be_r[g], 0, 0)),
            ],
            out_specs=pl.BlockSpec((BLK, HIDDEN), lambda g, be_r, na_r: (g, 0)),
        ),
        out_shape=jax.ShapeDtypeStruct((CAP, HIDDEN), jnp.float32),
    )(be, jnp.reshape(n_active, (1,)), xs, gate_rows.reshape(CAP, 1),
      W1, b1.reshape(E, 1, FF), W2, b2.reshape(E, 1, HIDDEN))

    out = _make_combine()(ys, dest0.reshape(NW, TPW), dest1.reshape(NW, TPW))
    return out.reshape(B, S, H)


# speedup vs baseline: 1.1510x; 1.1510x over previous
import functools

import jax
import jax.numpy as jnp
from jax import lax
from jax.experimental import pallas as pl
from jax.experimental.pallas import tpu as pltpu
from jax.experimental.pallas import tpu_sc as plsc

HIDDEN = 768
FF = 3072
E = 8
TOP_K = 2
T = 2048

BLK = 256
G = (T * TOP_K) // BLK + E
CAP = G * BLK

NW = 32
TPW = T // NW


def _router_body(x_ref, wg_ref, bg_ref, idx_ref, gates_ref):
    x = x_ref[...]
    logits = jax.lax.dot_general(
        x, wg_ref[...], (((1,), (0,)), ((), ())),
        preferred_element_type=jnp.float32) + bg_ref[...][None, :]
    col = jax.lax.broadcasted_iota(jnp.int32, (T, E), 1)
    m1 = jnp.max(logits, axis=1, keepdims=True)
    i1 = jnp.min(jnp.where(logits == m1, col, E), axis=1, keepdims=True)
    masked = jnp.where(col == i1, -jnp.inf, logits)
    m2 = jnp.max(masked, axis=1, keepdims=True)
    i2 = jnp.min(jnp.where(masked == m2, col, E), axis=1, keepdims=True)
    e2 = jnp.exp(m2 - m1)
    g1 = 1.0 / (1.0 + e2)
    g2 = e2 * g1
    idx_ref[...] = jnp.concatenate([i1, i2], axis=1)
    gates_ref[...] = jnp.concatenate([g1, g2], axis=1)


def _ffn_body(be_ref, na_ref, xs_ref, gr_ref, w1_ref, b1_ref, w2_ref,
              b2_ref, ys_ref):
    g = pl.program_id(0)

    @pl.when(g < na_ref[0])
    def _():
        h = jax.nn.gelu(jax.lax.dot_general(
            xs_ref[...], w1_ref[0], (((1,), (0,)), ((), ())),
            preferred_element_type=jnp.float32) + b1_ref[0])
        eo = jax.lax.dot_general(
            h, w2_ref[0], (((1,), (0,)), ((), ())),
            preferred_element_type=jnp.float32) + b2_ref[0]
        ys_ref[...] = eo * gr_ref[...]


def _make_dispatch():
    mesh = plsc.VectorSubcoreMesh(core_axis_name="c", subcore_axis_name="s")

    @functools.partial(
        pl.kernel, mesh=mesh,
        out_type=jax.ShapeDtypeStruct((CAP, HIDDEN), jnp.float32),
        scratch_types=[
            pltpu.VMEM((TPW, HIDDEN), jnp.float32),
            pltpu.VMEM((TPW,), jnp.int32),
            pltpu.VMEM((TPW,), jnp.int32),
            pltpu.SemaphoreType.DMA,
        ],
    )
    def dispatch(x_hbm, d0_hbm, d1_hbm, xs_hbm, xch, i0, i1, sem):
        wid = lax.axis_index("s") * 2 + lax.axis_index("c")
        base = wid * TPW
        pltpu.sync_copy(x_hbm.at[pl.ds(base, TPW)], xch)
        pltpu.sync_copy(d0_hbm.at[wid], i0)
        pltpu.sync_copy(d1_hbm.at[wid], i1)
        c0 = pltpu.async_copy(xch, xs_hbm.at[i0], sem)
        c1 = pltpu.async_copy(xch, xs_hbm.at[i1], sem)
        c0.wait()
        c1.wait()

    return dispatch


def _make_combine():
    mesh = plsc.VectorSubcoreMesh(core_axis_name="c", subcore_axis_name="s")

    @functools.partial(
        pl.kernel, mesh=mesh,
        out_type=jax.ShapeDtypeStruct((T, HIDDEN), jnp.float32),
        scratch_types=[
            pltpu.VMEM((TPW, HIDDEN), jnp.float32),
            pltpu.VMEM((TPW, HIDDEN), jnp.float32),
            pltpu.VMEM((TPW,), jnp.int32),
            pltpu.VMEM((TPW,), jnp.int32),
            pltpu.SemaphoreType.DMA,
        ],
    )
    def combine(ys_hbm, d0_hbm, d1_hbm, out_hbm, a, b, i0, i1, sem):
        wid = lax.axis_index("s") * 2 + lax.axis_index("c")
        base = wid * TPW
        pltpu.sync_copy(d0_hbm.at[wid], i0)
        pltpu.sync_copy(d1_hbm.at[wid], i1)
        c0 = pltpu.async_copy(ys_hbm.at[i0], a, sem)
        c1 = pltpu.async_copy(ys_hbm.at[i1], b, sem)
        c0.wait()
        c1.wait()

        def row_body(r, carry):
            for j in range(HIDDEN // 16):
                sl = pl.ds(j * 16, 16)
                a[r, sl] = a[r, sl] + b[r, sl]
            return carry

        lax.fori_loop(0, TPW, row_body, 0)
        pltpu.sync_copy(a, out_hbm.at[pl.ds(base, TPW)])

    return combine


def kernel(x, Wg, bg, W1, b1, W2, b2):
    B, S, H = x.shape
    x_flat = x.reshape(-1, H)

    top_idx, gates = pl.pallas_call(
        _router_body,
        grid=(1,),
        in_specs=[
            pl.BlockSpec((T, HIDDEN), lambda i: (0, 0)),
            pl.BlockSpec((HIDDEN, E), lambda i: (0, 0)),
            pl.BlockSpec((E,), lambda i: (0,)),
        ],
        out_specs=[
            pl.BlockSpec((T, TOP_K), lambda i: (0, 0)),
            pl.BlockSpec((T, TOP_K), lambda i: (0, 0)),
        ],
        out_shape=[
            jax.ShapeDtypeStruct((T, TOP_K), jnp.int32),
            jax.ShapeDtypeStruct((T, TOP_K), jnp.float32),
        ],
    )(x_flat, Wg, bg)

    oh = top_idx[..., None] == jnp.arange(E)[None, None, :]
    c = oh.sum(1).astype(jnp.int32)
    incl = jnp.cumsum(c, axis=0)
    excl = incl - c
    counts = incl[-1]
    blocks_e = (counts + BLK - 1) // BLK
    cumB = jnp.cumsum(blocks_e)
    row_start = (cumB - blocks_e) * BLK
    base = row_start[None, :] + excl
    dest0 = jnp.sum(jnp.where(oh[:, 0], base, 0), axis=-1).astype(jnp.int32)
    dest1 = jnp.sum(jnp.where(oh[:, 1], base, 0), axis=-1).astype(jnp.int32)
    n_active = cumB[-1]
    g_idx = jnp.arange(G, dtype=jnp.int32)
    be = jnp.clip((g_idx[:, None] >= cumB[None, :]).sum(-1), 0, E - 1)
    be = jnp.where(g_idx < n_active, be, be[n_active - 1]).astype(jnp.int32)
    gate_rows = (jnp.zeros((CAP,), jnp.float32)
                 .at[dest0].set(gates[:, 0])
                 .at[dest1].set(gates[:, 1]))

    xs = _make_dispatch()(x_flat, dest0.reshape(NW, TPW),
                          dest1.reshape(NW, TPW))

    ys = pl.pallas_call(
        _ffn_body,
        grid_spec=pltpu.PrefetchScalarGridSpec(
            num_scalar_prefetch=2,
            grid=(G,),
            in_specs=[
                pl.BlockSpec((BLK, HIDDEN), lambda g, be_r, na_r: (g, 0)),
                pl.BlockSpec((BLK, 1), lambda g, be_r, na_r: (g, 0)),
                pl.BlockSpec((1, HIDDEN, FF),
                             lambda g, be_r, na_r: (be_r[g], 0, 0)),
                pl.BlockSpec((1, 1, FF),
                             lambda g, be_r, na_r: (be_r[g], 0, 0)),
                pl.BlockSpec((1, FF, HIDDEN),
                             lambda g, be_r, na_r: (be_r[g], 0, 0)),
                pl.BlockSpec((1, 1, HIDDEN),
                             lambda g, be_r, na_r: (be_r[g], 0, 0)),
            ],
            out_specs=pl.BlockSpec((BLK, HIDDEN), lambda g, be_r, na_r: (g, 0)),
        ),
        out_shape=jax.ShapeDtypeStruct((CAP, HIDDEN), jnp.float32),
    )(be, jnp.reshape(n_active, (1,)), xs, gate_rows.reshape(CAP, 1),
      W1, b1.reshape(E, 1, FF), W2, b2.reshape(E, 1, HIDDEN))

    out = _make_combine()(ys, dest0.reshape(NW, TPW), dest1.reshape(NW, TPW))
    return out.reshape(B, S, H)


# speedup vs baseline: 1.3263x; 1.1523x over previous
import functools

import jax
import jax.numpy as jnp
from jax.experimental import pallas as pl
from jax.experimental.pallas import tpu as pltpu

HIDDEN = 768
FF = 3072
E = 8
TOP_K = 2
T = 2048

BLK = 256
G = (T * TOP_K) // BLK + E


def _router_body(x_ref, wg_ref, bg_ref, idx_ref, gates_ref):
    x = x_ref[...]
    logits = jax.lax.dot_general(
        x, wg_ref[...], (((1,), (0,)), ((), ())),
        preferred_element_type=jnp.float32) + bg_ref[...][None, :]
    col = jax.lax.broadcasted_iota(jnp.int32, (T, E), 1)
    m1 = jnp.max(logits, axis=1, keepdims=True)
    i1 = jnp.min(jnp.where(logits == m1, col, E), axis=1, keepdims=True)
    masked = jnp.where(col == i1, -jnp.inf, logits)
    m2 = jnp.max(masked, axis=1, keepdims=True)
    i2 = jnp.min(jnp.where(masked == m2, col, E), axis=1, keepdims=True)
    e2 = jnp.exp(m2 - m1)
    g1 = 1.0 / (1.0 + e2)
    g2 = e2 * g1
    idx_ref[...] = jnp.concatenate([i1, i2], axis=1)
    gates_ref[...] = jnp.concatenate([g1, g2], axis=1)


def _moe_body(be_ref, na_ref, x_ref, d0_ref, d1_ref, gw0_ref, gw1_ref,
              w1_ref, b1_ref, w2_ref, b2_ref, out_ref):
    g = pl.program_id(0)

    @pl.when(g == 0)
    def _():
        out_ref[...] = jnp.zeros_like(out_ref)

    @pl.when(g < na_ref[0])
    def _():
        row_ids = g * BLK + jax.lax.broadcasted_iota(jnp.int32, (BLK, T), 0)
        cmp0 = d0_ref[...] == row_ids
        cmp1 = d1_ref[...] == row_ids
        gmask = jnp.where(cmp0 | cmp1, 1.0, 0.0)
        rows = jax.lax.dot_general(
            gmask, x_ref[...], (((1,), (0,)), ((), ())),
            preferred_element_type=jnp.float32, precision=jax.lax.Precision.DEFAULT)
        h = jax.nn.gelu(jax.lax.dot_general(
            rows, w1_ref[0], (((1,), (0,)), ((), ())),
            preferred_element_type=jnp.float32, precision=jax.lax.Precision.DEFAULT) + b1_ref[0])
        eo = jax.lax.dot_general(
            h, w2_ref[0], (((1,), (0,)), ((), ())),
            preferred_element_type=jnp.float32, precision=jax.lax.Precision.DEFAULT) + b2_ref[0]
        gw = (jnp.where(cmp0, gw0_ref[...], 0.0)
              + jnp.where(cmp1, gw1_ref[...], 0.0))
        out_ref[...] += jax.lax.dot_general(
            gw, eo, (((0,), (0,)), ((), ())),
            preferred_element_type=jnp.float32, precision=jax.lax.Precision.DEFAULT)


def kernel(x, Wg, bg, W1, b1, W2, b2):
    B, S, H = x.shape
    x_flat = x.reshape(-1, H)

    top_idx, gates = pl.pallas_call(
        _router_body,
        grid=(1,),
        in_specs=[
            pl.BlockSpec((T, HIDDEN), lambda i: (0, 0)),
            pl.BlockSpec((HIDDEN, E), lambda i: (0, 0)),
            pl.BlockSpec((E,), lambda i: (0,)),
        ],
        out_specs=[
            pl.BlockSpec((T, TOP_K), lambda i: (0, 0)),
            pl.BlockSpec((T, TOP_K), lambda i: (0, 0)),
        ],
        out_shape=[
            jax.ShapeDtypeStruct((T, TOP_K), jnp.int32),
            jax.ShapeDtypeStruct((T, TOP_K), jnp.float32),
        ],
    )(x_flat, Wg, bg)

    oh = top_idx[..., None] == jnp.arange(E)[None, None, :]
    c = oh.sum(1).astype(jnp.int32)
    incl = jnp.cumsum(c, axis=0)
    excl = incl - c
    counts = incl[-1]
    blocks_e = (counts + BLK - 1) // BLK
    cumB = jnp.cumsum(blocks_e)
    row_start = (cumB - blocks_e) * BLK
    base = row_start[None, :] + excl
    dest0 = jnp.sum(jnp.where(oh[:, 0], base, 0), axis=-1).astype(jnp.int32)
    dest1 = jnp.sum(jnp.where(oh[:, 1], base, 0), axis=-1).astype(jnp.int32)
    n_active = cumB[-1]
    g_idx = jnp.arange(G, dtype=jnp.int32)
    be = jnp.clip((g_idx[:, None] >= cumB[None, :]).sum(-1), 0, E - 1)
    be = jnp.where(g_idx < n_active, be, be[n_active - 1]).astype(jnp.int32)

    out = pl.pallas_call(
        _moe_body,
        grid_spec=pltpu.PrefetchScalarGridSpec(
            num_scalar_prefetch=2,
            grid=(G,),
            in_specs=[
                pl.BlockSpec((T, HIDDEN), lambda g, be_r, na_r: (0, 0)),
                pl.BlockSpec((1, T), lambda g, be_r, na_r: (0, 0)),
                pl.BlockSpec((1, T), lambda g, be_r, na_r: (0, 0)),
                pl.BlockSpec((1, T), lambda g, be_r, na_r: (0, 0)),
                pl.BlockSpec((1, T), lambda g, be_r, na_r: (0, 0)),
                pl.BlockSpec((1, HIDDEN, FF),
                             lambda g, be_r, na_r: (be_r[g], 0, 0)),
                pl.BlockSpec((1, 1, FF),
                             lambda g, be_r, na_r: (be_r[g], 0, 0)),
                pl.BlockSpec((1, FF, HIDDEN),
                             lambda g, be_r, na_r: (be_r[g], 0, 0)),
                pl.BlockSpec((1, 1, HIDDEN),
                             lambda g, be_r, na_r: (be_r[g], 0, 0)),
            ],
            out_specs=pl.BlockSpec((T, HIDDEN), lambda g, be_r, na_r: (0, 0)),
        ),
        out_shape=jax.ShapeDtypeStruct((T, HIDDEN), jnp.float32),
    )(be, jnp.reshape(n_active, (1,)), x_flat,
      dest0.reshape(1, T), dest1.reshape(1, T),
      gates[:, 0].reshape(1, T), gates[:, 1].reshape(1, T),
      W1, b1.reshape(E, 1, FF), W2, b2.reshape(E, 1, HIDDEN))
    return out.reshape(B, S, H)
